# Initial kernel scaffold; baseline (speedup 1.0000x reference)
import jax
import jax.numpy as jnp
from jax.experimental import pallas as pl


def kernel(token_ids, table):
    raise NotImplementedError("write your pallas kernel here")


# speedup vs baseline: 1.7784x; 1.7784x over previous
import jax
import jax.numpy as jnp
from jax import lax
from jax.experimental import pallas as pl
from jax.experimental.pallas import tpu as pltpu
from jax.experimental.pallas import tpu_sc as plsc

BATCH = 4
SEQ_LEN = 4096
D_MODEL = 2560
NUM_TOKENS = BATCH * SEQ_LEN

NUM_CORES = 2
NUM_SUBCORES = 16
NUM_WORKERS = NUM_CORES * NUM_SUBCORES
IDS_PER_WORKER = NUM_TOKENS // NUM_WORKERS

C = 16
NUM_CHUNKS = IDS_PER_WORKER // C


def _gather_impl(table, ids_flat):
    mesh = plsc.VectorSubcoreMesh(core_axis_name="c", subcore_axis_name="s")

    @pl.kernel(
        out_type=jax.ShapeDtypeStruct((NUM_TOKENS, D_MODEL), jnp.float32),
        mesh=mesh,
        scratch_types=[
            pltpu.VMEM((IDS_PER_WORKER,), jnp.int32),
            pltpu.VMEM((C, D_MODEL), jnp.float32),
            pltpu.VMEM((C, D_MODEL), jnp.float32),
            pltpu.SemaphoreType.DMA,
            pltpu.SemaphoreType.DMA,
        ],
    )
    def gather_kernel(table_hbm, ids_hbm, out_hbm, idx_v, rows0, rows1,
                      sem0, sem1):
        wid = lax.axis_index("s") * NUM_CORES + lax.axis_index("c")
        base = wid * IDS_PER_WORKER
        pltpu.sync_copy(ids_hbm.at[pl.ds(base, IDS_PER_WORKER)], idx_v)

        bufs = (rows0, rows1)
        sems = (sem0, sem1)

        pltpu.async_copy(table_hbm.at[idx_v.at[pl.ds(0, C)]], rows0, sem0)
        pltpu.async_copy(table_hbm.at[idx_v.at[pl.ds(C, C)]], rows1, sem1)

        @pl.loop(0, NUM_CHUNKS, step=2)
        def _(i):
            for b in range(2):
                chunk = i + b
                buf = bufs[b]
                sem = sems[b]
                pltpu.make_async_copy(
                    table_hbm.at[idx_v.at[pl.ds(chunk * C, C)]], buf, sem
                ).wait()
                pltpu.sync_copy(buf, out_hbm.at[pl.ds(base + chunk * C, C)])

                @pl.when(chunk + 2 < NUM_CHUNKS)
                def _():
                    pltpu.async_copy(
                        table_hbm.at[idx_v.at[pl.ds((chunk + 2) * C, C)]],
                        buf, sem,
                    )

    return gather_kernel(table, ids_flat)


def kernel(token_ids, table):
    ids_flat = token_ids.reshape(NUM_TOKENS).astype(jnp.int32)
    out = _gather_impl(table, ids_flat)
    return out.reshape(BATCH, SEQ_LEN, D_MODEL)


# speedup vs baseline: 1.7816x; 1.0018x over previous
import jax
import jax.numpy as jnp
from jax import lax
from jax.experimental import pallas as pl
from jax.experimental.pallas import tpu as pltpu
from jax.experimental.pallas import tpu_sc as plsc

BATCH = 4
SEQ_LEN = 4096
D_MODEL = 2560
NUM_TOKENS = BATCH * SEQ_LEN

NUM_CORES = 2
NUM_SUBCORES = 16
NUM_WORKERS = NUM_CORES * NUM_SUBCORES
IDS_PER_WORKER = NUM_TOKENS // NUM_WORKERS

C = 8
NBUF = 4
NUM_CHUNKS = IDS_PER_WORKER // C


def _gather_impl(table, ids_flat):
    mesh = plsc.VectorSubcoreMesh(core_axis_name="c", subcore_axis_name="s")

    @pl.kernel(
        out_type=jax.ShapeDtypeStruct((NUM_TOKENS, D_MODEL), jnp.float32),
        mesh=mesh,
        scratch_types=(
            [pltpu.VMEM((IDS_PER_WORKER,), jnp.int32)]
            + [pltpu.VMEM((C, D_MODEL), jnp.float32) for _ in range(NBUF)]
            + [pltpu.SemaphoreType.DMA for _ in range(2 * NBUF)]
        ),
    )
    def gather_kernel(table_hbm, ids_hbm, out_hbm, idx_v, *bufs_sems):
        bufs = bufs_sems[:NBUF]
        gsems = bufs_sems[NBUF:2 * NBUF]
        osems = bufs_sems[2 * NBUF:]

        wid = lax.axis_index("s") * NUM_CORES + lax.axis_index("c")
        base = wid * IDS_PER_WORKER
        pltpu.sync_copy(ids_hbm.at[pl.ds(base, IDS_PER_WORKER)], idx_v)

        def gather_start(chunk, b):
            pltpu.async_copy(
                table_hbm.at[idx_v.at[pl.ds(chunk * C, C)]], bufs[b], gsems[b]
            )

        for b in range(NBUF - 1):
            gather_start(b, b)

        @pl.loop(0, NUM_CHUNKS, step=NBUF)
        def _(i):
            for b in range(NBUF):
                chunk = i + b
                bp = (b - 1) % NBUF
                nxt = chunk + NBUF - 1

                @pl.when(jnp.logical_and(chunk >= 1, nxt < NUM_CHUNKS))
                def _():
                    pltpu.make_async_copy(
                        bufs[bp], out_hbm.at[pl.ds(base, C)], osems[bp]
                    ).wait()

                @pl.when(nxt < NUM_CHUNKS)
                def _():
                    gather_start(nxt, bp)

                pltpu.make_async_copy(
                    table_hbm.at[idx_v.at[pl.ds(chunk * C, C)]],
                    bufs[b], gsems[b],
                ).wait()
                pltpu.async_copy(
                    bufs[b], out_hbm.at[pl.ds(base + chunk * C, C)], osems[b]
                )

        for b in range(NBUF):
            pltpu.make_async_copy(
                bufs[b], out_hbm.at[pl.ds(base, C)], osems[b]
            ).wait()

    return gather_kernel(table, ids_flat)


def kernel(token_ids, table):
    ids_flat = token_ids.reshape(NUM_TOKENS).astype(jnp.int32)
    out = _gather_impl(table, ids_flat)
    return out.reshape(BATCH, SEQ_LEN, D_MODEL)
